# baseline (device time: 460823 ns/iter reference)
import jax
import jax.numpy as jnp
from jax import lax
from jax.experimental import pallas as pl
from jax.experimental.pallas import tpu as pltpu

N_DEV = 4
M, K, N = 4096, 4096, 8192
KS = K // N_DEV
MH = M // 2
KH = KS // 2
NQ = 4
QN = N // NQ
BM, BN = 512, QN


def _body(
    scale_ref, x_ref, w_ref, o_ref,
    gx, gw, stage,
    loc_sems, sx_sems, rx_sems, sw_sems, rw_sems, out_sems,
):
    me = lax.axis_index("i")
    right = (me + 1) % N_DEV
    left = (me - 1) % N_DEV

    cp_x = pltpu.make_async_copy(
        x_ref, gx.at[:, pl.ds(me * KS, KS)], loc_sems.at[0]
    )
    cp_w = pltpu.make_async_copy(
        w_ref, gw.at[:, pl.ds(me * KS, KS), :], loc_sems.at[1]
    )
    cp_x.start()
    cp_w.start()

    def x_slc(s, d):
        return gx.at[pl.ds(d * MH, MH), pl.ds(s * KS, KS)]

    def w_slc(q, s, d):
        return gw.at[q, pl.ds(s * KS + d * KH, KH), :]

    def hop(h, slc, src0, ssem_at, rsem_at):
        sR = (me - h) % N_DEV
        sL = (me + h) % N_DEV
        sends, recvs = [], []
        for d, s, tgt in [(0, sR, right), (1, sL, left)]:
            src = src0(d) if h == 0 else slc(s, d)
            rdma = pltpu.make_async_remote_copy(
                src_ref=src,
                dst_ref=slc(s, d),
                send_sem=ssem_at(d, h),
                recv_sem=rsem_at(d, h),
                device_id=(tgt,),
                device_id_type=pl.DeviceIdType.MESH,
            )
            rdma.start()
            sends.append(rdma)
        for d, r in [(0, (me - h - 1) % N_DEV), (1, (me + h + 1) % N_DEV)]:
            recvs.append(
                pltpu.make_async_remote_copy(
                    src_ref=slc(r, d),
                    dst_ref=slc(r, d),
                    send_sem=ssem_at(d, h),
                    recv_sem=rsem_at(d, h),
                    device_id=(left,),
                    device_id_type=pl.DeviceIdType.MESH,
                )
            )
        return sends, recvs

    def x_hop(h):
        return hop(
            h, x_slc, lambda d: x_ref.at[pl.ds(d * MH, MH)],
            lambda d, hh: sx_sems.at[d, hh],
            lambda d, hh: rx_sems.at[d, hh],
        )

    def w_hop(q, h):
        return hop(
            h,
            lambda s, d: w_slc(q, s, d),
            lambda d: w_ref.at[q, pl.ds(d * KH, KH), :],
            lambda d, hh: sw_sems.at[q, d, hh],
            lambda d, hh: rw_sems.at[q, d, hh],
        )

    scale = scale_ref[0]
    pending = []

    def tile(q, t):
        i = (q * 8 + t) % 2
        if len(pending) >= 2:
            pending.pop(0).wait()
        y = jnp.dot(
            gx[pl.ds(t * BM, BM), :],
            gw[q],
            preferred_element_type=jnp.float32,
        ) * scale
        stage[i] = y * jax.nn.sigmoid(y)
        cp = pltpu.make_async_copy(
            stage.at[i],
            o_ref.at[pl.ds(t * BM, BM), pl.ds(q * QN, QN)],
            out_sems.at[i],
        )
        cp.start()
        pending.append(cp)

    all_sends = []

    for h in range(N_DEV - 1):
        s, r = x_hop(h)
        all_sends += s
        for rc in r:
            rc.wait_recv()
    for h in range(N_DEV - 1):
        s, r = w_hop(0, h)
        all_sends += s
        for rc in r:
            rc.wait_recv()
    cp_x.wait()
    cp_w.wait()

    for q in range(1, NQ):
        plan = [(0, [0, 1]), (1, [2, 3]), (2, [4, 5, 6, 7])]
        for h, tiles in plan:
            s, r = w_hop(q, h)
            all_sends += s
            for t in tiles:
                tile(q - 1, t)
            for rc in r:
                rc.wait_recv()

    for t in range(8):
        tile(NQ - 1, t)

    for cp in pending:
        cp.wait()
    for rdma in all_sends:
        rdma.wait_send()


def kernel(x, w_mat, scale_x, scale_w):
    x8 = x.astype(jnp.float8_e4m3fn)
    w8 = jnp.transpose(
        w_mat.astype(jnp.float8_e5m2).reshape(KS, NQ, QN), (1, 0, 2)
    )
    scale = (scale_x * scale_w).astype(jnp.float32)

    return pl.pallas_call(
        _body,
        out_shape=jax.ShapeDtypeStruct((M, N), jnp.float32),
        in_specs=[
            pl.BlockSpec(memory_space=pltpu.SMEM),
            pl.BlockSpec(memory_space=pl.ANY),
            pl.BlockSpec(memory_space=pl.ANY),
        ],
        out_specs=pl.BlockSpec(memory_space=pl.ANY),
        scratch_shapes=[
            pltpu.VMEM((M, K), jnp.float8_e4m3fn),
            pltpu.VMEM((NQ, K, QN), jnp.float8_e5m2),
            pltpu.VMEM((2, BM, BN), jnp.float32),
            pltpu.SemaphoreType.DMA((2,)),
            pltpu.SemaphoreType.DMA((2, N_DEV - 1)),
            pltpu.SemaphoreType.DMA((2, N_DEV - 1)),
            pltpu.SemaphoreType.DMA((NQ, 2, N_DEV - 1)),
            pltpu.SemaphoreType.DMA((NQ, 2, N_DEV - 1)),
            pltpu.SemaphoreType.DMA((2,)),
        ],
        compiler_params=pltpu.CompilerParams(
            vmem_limit_bytes=100 * 1024 * 1024,
        ),
    )(scale, x8, w8)


# device time: 410117 ns/iter; 1.1236x vs baseline; 1.1236x over previous
import jax
import jax.numpy as jnp
from jax import lax
from jax.experimental import pallas as pl
from jax.experimental.pallas import tpu as pltpu

N_DEV = 4
M, K, N = 4096, 4096, 8192
KS = K // N_DEV

BM, BN = 1024, 2048
MH, KH = M // 2, KS // 2


def _ag_body(x_ref, w_ref, gx_ref, gw_ref, local_sems, send_sems, recv_sems):
    me = lax.axis_index("i")
    right = (me + 1) % N_DEV
    left = (me - 1) % N_DEV

    cp_x = pltpu.make_async_copy(
        x_ref, gx_ref.at[:, pl.ds(me * KS, KS)], local_sems.at[0]
    )
    cp_w = pltpu.make_async_copy(
        w_ref, gw_ref.at[pl.ds(me * KS, KS)], local_sems.at[1]
    )
    cp_x.start()
    cp_w.start()

    def x_slice(s, d):
        return gx_ref.at[pl.ds(0 if d == 0 else MH, MH), pl.ds(s * KS, KS)]

    def w_slice(s, d):
        return gw_ref.at[pl.ds(s * KS + (0 if d == 0 else KH), KH)]

    for h in range(N_DEV - 1):
        sR = (me - h) % N_DEV
        sL = (me + h) % N_DEV
        sends = []
        for t, slc in enumerate([x_slice, w_slice]):
            for d, s, tgt in [(0, sR, right), (1, sL, left)]:
                if h == 0:
                    inp, hh = (x_ref, MH) if t == 0 else (w_ref, KH)
                    src = inp.at[pl.ds(0 if d == 0 else hh, hh)]
                else:
                    src = slc(s, d)
                rdma = pltpu.make_async_remote_copy(
                    src_ref=src,
                    dst_ref=slc(s, d),
                    send_sem=send_sems.at[t, d, h],
                    recv_sem=recv_sems.at[t, d, h],
                    device_id=(tgt,),
                    device_id_type=pl.DeviceIdType.MESH,
                )
                rdma.start()
                sends.append(rdma)
        for t, slc in enumerate([x_slice, w_slice]):
            for d, r in [(0, (me - h - 1) % N_DEV), (1, (me + h + 1) % N_DEV)]:
                recv = pltpu.make_async_remote_copy(
                    src_ref=slc(r, d),
                    dst_ref=slc(r, d),
                    send_sem=send_sems.at[t, d, h],
                    recv_sem=recv_sems.at[t, d, h],
                    device_id=(left,),
                    device_id_type=pl.DeviceIdType.MESH,
                )
                recv.wait_recv()
        for rdma in sends:
            rdma.wait_send()

    cp_x.wait()
    cp_w.wait()


def _gemm_body(scale_ref, gx_ref, gw_ref, o_ref):
    m = pl.program_id(1)
    y = jnp.dot(
        gx_ref[pl.ds(m * BM, BM), :],
        gw_ref[...],
        preferred_element_type=jnp.float32,
    ) * scale_ref[0]
    o_ref[...] = y * jax.nn.sigmoid(y)


def kernel(x, w_mat, scale_x, scale_w):
    x8 = x.astype(jnp.float8_e4m3fn)
    w8 = w_mat.astype(jnp.float8_e5m2)

    gx, gw = pl.pallas_call(
        _ag_body,
        out_shape=(
            jax.ShapeDtypeStruct((M, K), jnp.float8_e4m3fn),
            jax.ShapeDtypeStruct((K, N), jnp.float8_e5m2),
        ),
        in_specs=[
            pl.BlockSpec(memory_space=pl.ANY),
            pl.BlockSpec(memory_space=pl.ANY),
        ],
        out_specs=(
            pl.BlockSpec(memory_space=pl.ANY),
            pl.BlockSpec(memory_space=pl.ANY),
        ),
        scratch_shapes=[
            pltpu.SemaphoreType.DMA((2,)),
            pltpu.SemaphoreType.DMA((2, 2, N_DEV - 1)),
            pltpu.SemaphoreType.DMA((2, 2, N_DEV - 1)),
        ],
    )(x8, w8)

    scale = (scale_x * scale_w).astype(jnp.float32)

    return pl.pallas_call(
        _gemm_body,
        grid=(N // BN, M // BM),
        in_specs=[
            pl.BlockSpec(memory_space=pltpu.SMEM),
            pl.BlockSpec(memory_space=pltpu.VMEM),
            pl.BlockSpec((K, BN), lambda n, m: (0, n)),
        ],
        out_specs=pl.BlockSpec((BM, BN), lambda n, m: (m, n)),
        out_shape=jax.ShapeDtypeStruct((M, N), jnp.float32),
        compiler_params=pltpu.CompilerParams(
            dimension_semantics=("parallel", "parallel"),
            vmem_limit_bytes=100 * 1024 * 1024,
        ),
    )(scale, gx, gw)
